# Initial kernel scaffold; baseline (speedup 1.0000x reference)
import jax
import jax.numpy as jnp
from jax.experimental import pallas as pl


def kernel(embeddings, codebooks):
    raise NotImplementedError("write your pallas kernel here")


# speedup vs baseline: 1.4287x; 1.4287x over previous
import functools

import jax
import jax.numpy as jnp
from jax import lax
from jax.experimental import pallas as pl
from jax.experimental.pallas import tpu as pltpu, tpu_sc as plsc

NSUB = 8
KCB = 8192
SD = 64
AUG = 128
NTOK = 8192

TN = 512
TK = 2048
HALF = 4096

CN = 128
QUARTERS = 4


def _bf16_round(v):
    u = lax.bitcast_convert_type(v, jnp.uint32)
    r = (u + jnp.uint32(0x7FFF) + ((u >> 16) & jnp.uint32(1))) & jnp.uint32(0xFFFF0000)
    return lax.bitcast_convert_type(r, jnp.float32)


def _pq_argmin_body(cb_ref, xt_ref, x2_ref, out_ref):
    s = pl.program_id(0)
    xt = xt_ref[0]
    x2r = x2_ref[0]

    def chunk_min(kt):
        c = cb_ref[0, pl.ds(kt * TK, TK), 0:SD]
        c2col = cb_ref[0, pl.ds(kt * TK, TK), SD:SD + 1]
        xc2 = lax.dot_general(
            c, xt, (((1,), (0,)), ((), ())),
            preferred_element_type=jnp.float32,
            precision=lax.Precision.DEFAULT,
        )
        d2 = (c2col + x2r) + xc2
        mv = jnp.min(d2, axis=0, keepdims=True)
        rows = lax.broadcasted_iota(jnp.int32, (TK, TN), 0)
        mi = jnp.min(jnp.where(d2 == mv, rows, TK), axis=0, keepdims=True)
        return mv, mi + kt * TK

    def merge(a, b):
        upd = b[0] < a[0]
        return jnp.where(upd, b[0], a[0]), jnp.where(upd, b[1], a[1])

    halves = []
    for h in range(2):
        best = chunk_min(h * (HALF // TK))
        for j in range(1, HALF // TK):
            best = merge(best, chunk_min(h * (HALF // TK) + j))
        halves.append(best)
    (b0, i0), (b1, i1) = halves

    s0 = jnp.sqrt(jnp.maximum(b0, 0.0))
    s1 = jnp.sqrt(jnp.maximum(b1, 0.0))
    take = s1 < _bf16_round(s0)
    out_ref[0] = jnp.where(take, i1, i0) + s * KCB


_pq_indices = pl.pallas_call(
    _pq_argmin_body,
    grid=(NSUB, NTOK // TN),
    in_specs=[
        pl.BlockSpec((1, KCB, AUG), lambda s, nt: (s, 0, 0)),
        pl.BlockSpec((1, SD, TN), lambda s, nt: (s, 0, nt)),
        pl.BlockSpec((1, 1, TN), lambda s, nt: (s, 0, nt)),
    ],
    out_specs=pl.BlockSpec((1, 1, TN), lambda s, nt: (s, 0, nt)),
    out_shape=jax.ShapeDtypeStruct((NSUB, 1, NTOK), jnp.int32),
)


@functools.cache
def _make_sc_gather():
    @functools.partial(
        pl.kernel,
        out_type=jax.ShapeDtypeStruct((NSUB * NTOK, AUG), jnp.float32),
        mesh=plsc.VectorSubcoreMesh(core_axis_name="c", subcore_axis_name="s"),
        scratch_types=[
            pltpu.VMEM((CN,), jnp.int32),
            pltpu.VMEM((CN, AUG), jnp.float32),
            pltpu.SemaphoreType.DMA,
        ],
    )
    def _sc_gather(table_hbm, fidx_hbm, out_hbm, idx_v, rows_v, sem):
        wid = lax.axis_index("s") * 2 + lax.axis_index("c")
        sub = wid // QUARTERS
        q = wid % QUARTERS
        tpq = NTOK // QUARTERS
        for i in range(tpq // CN):
            r0 = sub * NTOK + q * tpq + i * CN
            pltpu.sync_copy(fidx_hbm.at[pl.ds(r0, CN)], idx_v)
            pltpu.async_copy(table_hbm.at[idx_v], rows_v, sem).wait()
            pltpu.sync_copy(rows_v, out_hbm.at[pl.ds(r0, CN)])

    return _sc_gather


def _prep(embeddings, codebooks):
    b, s_len, d = embeddings.shape
    ns, k, sd = codebooks.shape
    n = b * s_len
    x = embeddings.reshape(n, ns, sd)
    xt = -2.0 * jnp.transpose(x, (1, 2, 0))
    x2 = jnp.sum(x * x, axis=-1).T.reshape(ns, 1, n)
    c2 = jnp.sum(codebooks * codebooks, axis=-1)
    cb_aug = jnp.concatenate(
        [codebooks, c2[:, :, None],
         jnp.zeros((ns, k, AUG - sd - 1), jnp.float32)], axis=2)
    return cb_aug, xt, x2


def kernel(embeddings, codebooks):
    b, s_len, d = embeddings.shape
    ns, k, sd = codebooks.shape
    n = b * s_len

    cb_aug, xt, x2 = _prep(embeddings, codebooks)
    fidx = _pq_indices(cb_aug, xt, x2)
    rows = _make_sc_gather()(cb_aug.reshape(ns * k, AUG), fidx.reshape(-1))
    quant = jnp.transpose(rows[:, :sd].reshape(ns, n, sd), (1, 0, 2))
    return quant.reshape(b, s_len, d)
